# baseline (device time: 22115 ns/iter reference)
import jax
import jax.numpy as jnp
from jax import lax
from jax.experimental import pallas as pl
from jax.experimental.pallas import tpu as pltpu

N_DEV = 4
B, SQ, SKV, HQ, DH = 2, 128, 512, 16, 64
H_LOC = HQ // N_DEV
KV_SRC = 2
SKV_LOC = SKV // N_DEV
SKV_EFF = KV_SRC * SKV_LOC
D_MODEL = 512
D_LOC = H_LOC * DH
D_CH = D_MODEL // N_DEV
WINDOW = 128
NEG = -1e9


def kernel(x, Wq, K_ext, V_ext, Wo):
    K2 = K_ext.reshape(B, SKV_LOC, HQ * DH).astype(jnp.bfloat16)
    V2 = V_ext.reshape(B, SKV_LOC, HQ * DH).astype(jnp.bfloat16)

    def body(x_ref, wq_ref, k_ref, v_ref, wo_ref, out_ref,
             kbuf, vbuf, mypart, rsbuf, agbuf,
             kv_send_sems, kv_recv_sems, loc_sems,
             rs_send_sems, rs_recv_sems, ag_send_sems, ag_recv_sems):
        my = lax.axis_index("i")

        barrier = pltpu.get_barrier_semaphore()
        for off in range(1, N_DEV):
            pl.semaphore_signal(
                barrier, inc=1,
                device_id=((my + off) % N_DEV,),
                device_id_type=pl.DeviceIdType.MESH,
            )
        pl.semaphore_wait(barrier, N_DEV - 1)

        for s in range(KV_SRC):
            @pl.when(my == s)
            def _(s=s):
                for t, (src, dst) in enumerate(((k_ref, kbuf), (v_ref, vbuf))):
                    for d in range(N_DEV):
                        if d == s:
                            continue
                        pltpu.make_async_remote_copy(
                            src_ref=src.at[:, :, pl.ds(d * D_LOC, D_LOC)],
                            dst_ref=dst.at[s],
                            send_sem=kv_send_sems.at[d, t],
                            recv_sem=kv_recv_sems.at[s, t],
                            device_id=(d,),
                            device_id_type=pl.DeviceIdType.MESH,
                        ).start()
                for t, (src, dst) in enumerate(((k_ref, kbuf), (v_ref, vbuf))):
                    pltpu.make_async_copy(
                        src.at[:, :, pl.ds(s * D_LOC, D_LOC)],
                        dst.at[s], loc_sems.at[t],
                    ).start()

        q_all = [
            [jnp.dot(x_ref[b], wq_ref[:, h * DH:(h + 1) * DH],
                     preferred_element_type=jnp.float32)
             for h in range(H_LOC)]
            for b in range(B)
        ]

        for s in range(KV_SRC):
            @pl.when(my == s)
            def _(s=s):
                for t, (src, dst) in enumerate(((k_ref, kbuf), (v_ref, vbuf))):
                    pltpu.make_async_copy(
                        src.at[:, :, pl.ds(s * D_LOC, D_LOC)],
                        dst.at[s], loc_sems.at[t],
                    ).wait()

        for s in range(KV_SRC):
            @pl.when(my != s)
            def _(s=s):
                pltpu.make_async_remote_copy(
                    src_ref=k_ref.at[:, :, pl.ds(0, D_LOC)],
                    dst_ref=kbuf.at[s],
                    send_sem=kv_send_sems.at[0, 0],
                    recv_sem=kv_recv_sems.at[s, 0],
                    device_id=(s,),
                    device_id_type=pl.DeviceIdType.MESH,
                ).wait_recv()

        qi = lax.broadcasted_iota(jnp.int32, (SQ, SKV_EFF), 0)
        ki = lax.broadcasted_iota(jnp.int32, (SQ, SKV_EFF), 1)
        mask = ki <= qi + WINDOW

        w_all = [[None] * H_LOC for _ in range(B)]
        for b in range(B):
            for h in range(H_LOC):
                kc = kbuf[:, b, :, h * DH:(h + 1) * DH].reshape(
                    SKV_EFF, DH).astype(jnp.float32)
                scores = lax.dot_general(
                    q_all[b][h], kc, (((1,), (1,)), ((), ())),
                    preferred_element_type=jnp.float32) * 0.125
                scores = jnp.where(mask, scores, NEG)
                m = jnp.max(scores, axis=1, keepdims=True)
                w = jnp.exp(scores - m)
                w_all[b][h] = w / jnp.sum(w, axis=1, keepdims=True)

        for s in range(KV_SRC):
            @pl.when(my != s)
            def _(s=s):
                pltpu.make_async_remote_copy(
                    src_ref=k_ref.at[:, :, pl.ds(0, D_LOC)],
                    dst_ref=vbuf.at[s],
                    send_sem=kv_send_sems.at[0, 1],
                    recv_sem=kv_recv_sems.at[s, 1],
                    device_id=(s,),
                    device_id_type=pl.DeviceIdType.MESH,
                ).wait_recv()

        for b in range(B):
            ctx_cat = jnp.concatenate(
                [jnp.dot(w_all[b][h],
                         vbuf[:, b, :, h * DH:(h + 1) * DH].reshape(
                             SKV_EFF, DH).astype(jnp.float32),
                         preferred_element_type=jnp.float32)
                 for h in range(H_LOC)],
                axis=1)
            for j in range(N_DEV):
                chunk = jnp.dot(ctx_cat, wo_ref[:, j * D_CH:(j + 1) * D_CH],
                                preferred_element_type=jnp.float32)
                mypart[b, :, j * D_CH:(j + 1) * D_CH] = chunk.astype(
                    jnp.bfloat16)
                for d in range(N_DEV):
                    @pl.when(my == d)
                    def _(d=d, b=b, j=j, chunk=chunk):
                        if j == d:
                            rsbuf[d, b] = chunk.astype(jnp.bfloat16)
                        else:
                            pltpu.make_async_remote_copy(
                                src_ref=mypart.at[b, :, pl.ds(j * D_CH, D_CH)],
                                dst_ref=rsbuf.at[d, b],
                                send_sem=rs_send_sems.at[j, b],
                                recv_sem=rs_recv_sems.at[d, b],
                                device_id=(j,),
                                device_id_type=pl.DeviceIdType.MESH,
                            ).start()

        for s in range(KV_SRC):
            @pl.when(my == s)
            def _(s=s):
                for d in range(N_DEV):
                    if d == s:
                        continue
                    for t, (src, dst) in enumerate(((k_ref, kbuf), (v_ref, vbuf))):
                        pltpu.make_async_remote_copy(
                            src_ref=src.at[:, :, pl.ds(d * D_LOC, D_LOC)],
                            dst_ref=dst.at[s],
                            send_sem=kv_send_sems.at[d, t],
                            recv_sem=kv_recv_sems.at[s, t],
                            device_id=(d,),
                            device_id_type=pl.DeviceIdType.MESH,
                        ).wait_send()

        for b in range(B):
            for s in range(N_DEV):
                @pl.when(my != s)
                def _(s=s, b=b):
                    pltpu.make_async_remote_copy(
                        src_ref=mypart.at[b, :, pl.ds(0, D_CH)],
                        dst_ref=rsbuf.at[s, b],
                        send_sem=rs_send_sems.at[s, b],
                        recv_sem=rs_recv_sems.at[s, b],
                        device_id=(s,),
                        device_id_type=pl.DeviceIdType.MESH,
                    ).wait_recv()
            red = (rsbuf[0, b].astype(jnp.float32)
                   + rsbuf[1, b].astype(jnp.float32)
                   + rsbuf[2, b].astype(jnp.float32)
                   + rsbuf[3, b].astype(jnp.float32))
            for j in range(N_DEV):
                @pl.when(my == j)
                def _(j=j, b=b, red=red):
                    agbuf[j, b] = red.astype(jnp.bfloat16)
                    for peer in range(N_DEV):
                        if peer == j:
                            continue
                        pltpu.make_async_remote_copy(
                            src_ref=agbuf.at[j, b],
                            dst_ref=agbuf.at[j, b],
                            send_sem=ag_send_sems.at[peer, b],
                            recv_sem=ag_recv_sems.at[j, b],
                            device_id=(peer,),
                            device_id_type=pl.DeviceIdType.MESH,
                        ).start()

        for b in range(B):
            for j in range(N_DEV):
                @pl.when(my != j)
                def _(j=j, b=b):
                    pltpu.make_async_remote_copy(
                        src_ref=agbuf.at[j, b], dst_ref=agbuf.at[j, b],
                        send_sem=ag_send_sems.at[j, b],
                        recv_sem=ag_recv_sems.at[j, b],
                        device_id=(j,),
                        device_id_type=pl.DeviceIdType.MESH,
                    ).wait_recv()
            for j in range(N_DEV):
                out_ref[b, :, j * D_CH:(j + 1) * D_CH] = agbuf[j, b].astype(
                    jnp.float32)

        for d in range(N_DEV):
            @pl.when(my == d)
            def _(d=d):
                for b in range(B):
                    for j in range(N_DEV):
                        if j == d:
                            continue
                        pltpu.make_async_remote_copy(
                            src_ref=mypart.at[b, :, pl.ds(j * D_CH, D_CH)],
                            dst_ref=rsbuf.at[d, b],
                            send_sem=rs_send_sems.at[j, b],
                            recv_sem=rs_recv_sems.at[d, b],
                            device_id=(j,),
                            device_id_type=pl.DeviceIdType.MESH,
                        ).wait_send()
                        pltpu.make_async_remote_copy(
                            src_ref=agbuf.at[d, b], dst_ref=agbuf.at[d, b],
                            send_sem=ag_send_sems.at[j, b],
                            recv_sem=ag_recv_sems.at[d, b],
                            device_id=(j,),
                            device_id_type=pl.DeviceIdType.MESH,
                        ).wait_send()

    return pl.pallas_call(
        body,
        out_shape=jax.ShapeDtypeStruct((B, SQ, D_MODEL), jnp.float32),
        in_specs=[
            pl.BlockSpec(memory_space=pltpu.VMEM),
            pl.BlockSpec(memory_space=pltpu.VMEM),
            pl.BlockSpec(memory_space=pltpu.MemorySpace.HBM),
            pl.BlockSpec(memory_space=pltpu.MemorySpace.HBM),
            pl.BlockSpec(memory_space=pltpu.VMEM),
        ],
        out_specs=pl.BlockSpec(memory_space=pltpu.VMEM),
        scratch_shapes=[
            pltpu.VMEM((KV_SRC, B, SKV_LOC, D_LOC), jnp.bfloat16),
            pltpu.VMEM((KV_SRC, B, SKV_LOC, D_LOC), jnp.bfloat16),
            pltpu.VMEM((B, SQ, D_MODEL), jnp.bfloat16),
            pltpu.VMEM((N_DEV, B, SQ, D_CH), jnp.bfloat16),
            pltpu.VMEM((N_DEV, B, SQ, D_CH), jnp.bfloat16),
            pltpu.SemaphoreType.DMA((N_DEV, 2)),
            pltpu.SemaphoreType.DMA((KV_SRC, 2)),
            pltpu.SemaphoreType.DMA((2,)),
            pltpu.SemaphoreType.DMA((N_DEV, B)),
            pltpu.SemaphoreType.DMA((N_DEV, B)),
            pltpu.SemaphoreType.DMA((N_DEV, B)),
            pltpu.SemaphoreType.DMA((N_DEV, B)),
        ],
        compiler_params=pltpu.CompilerParams(collective_id=0),
    )(x, Wq, K2, V2, Wo)


# device time: 21723 ns/iter; 1.0180x vs baseline; 1.0180x over previous
import jax
import jax.numpy as jnp
from jax import lax
from jax.experimental import pallas as pl
from jax.experimental.pallas import tpu as pltpu

N_DEV = 4
B, SQ, SKV, HQ, DH = 2, 128, 512, 16, 64
H_LOC = HQ // N_DEV
KV_SRC = 2
SKV_LOC = SKV // N_DEV
SKV_EFF = KV_SRC * SKV_LOC
D_MODEL = 512
D_LOC = H_LOC * DH
D_CH = D_MODEL // N_DEV
WINDOW = 128
NEG = -1e9


def kernel(x, Wq, K_ext, V_ext, Wo):
    K2 = K_ext.reshape(B, SKV_LOC, HQ * DH).astype(jnp.bfloat16)
    V2 = V_ext.reshape(B, SKV_LOC, HQ * DH).astype(jnp.bfloat16)

    def body(x_ref, wq_ref, k_ref, v_ref, wo_ref, out_ref,
             kbuf, vbuf, mypart, rsbuf, agbuf,
             kv_send_sems, kv_recv_sems, loc_sems,
             rs_send_sems, rs_recv_sems, ag_send_sems, ag_recv_sems):
        my = lax.axis_index("i")

        barrier = pltpu.get_barrier_semaphore()
        for off in range(1, N_DEV):
            pl.semaphore_signal(
                barrier, inc=1,
                device_id=((my + off) % N_DEV,),
                device_id_type=pl.DeviceIdType.MESH,
            )
        pl.semaphore_wait(barrier, N_DEV - 1)

        for s in range(KV_SRC):
            @pl.when(my == s)
            def _(s=s):
                for t, (src, dst) in enumerate(((k_ref, kbuf), (v_ref, vbuf))):
                    for d in range(N_DEV):
                        if d == s:
                            continue
                        pltpu.make_async_remote_copy(
                            src_ref=src.at[:, :, pl.ds(d * D_LOC, D_LOC)],
                            dst_ref=dst.at[s],
                            send_sem=kv_send_sems.at[d, t],
                            recv_sem=kv_recv_sems.at[s, t],
                            device_id=(d,),
                            device_id_type=pl.DeviceIdType.MESH,
                        ).start()
                for t, (src, dst) in enumerate(((k_ref, kbuf), (v_ref, vbuf))):
                    pltpu.make_async_copy(
                        src.at[:, :, pl.ds(s * D_LOC, D_LOC)],
                        dst.at[s], loc_sems.at[t],
                    ).start()

        q_all = [
            [jnp.dot(x_ref[b], wq_ref[:, h * DH:(h + 1) * DH],
                     preferred_element_type=jnp.float32)
             for h in range(H_LOC)]
            for b in range(B)
        ]

        for s in range(KV_SRC):
            @pl.when(my == s)
            def _(s=s):
                for t, (src, dst) in enumerate(((k_ref, kbuf), (v_ref, vbuf))):
                    pltpu.make_async_copy(
                        src.at[:, :, pl.ds(s * D_LOC, D_LOC)],
                        dst.at[s], loc_sems.at[t],
                    ).wait()

        for s in range(KV_SRC):
            @pl.when(my != s)
            def _(s=s):
                pltpu.make_async_remote_copy(
                    src_ref=k_ref.at[:, :, pl.ds(0, D_LOC)],
                    dst_ref=kbuf.at[s],
                    send_sem=kv_send_sems.at[0, 0],
                    recv_sem=kv_recv_sems.at[s, 0],
                    device_id=(s,),
                    device_id_type=pl.DeviceIdType.MESH,
                ).wait_recv()

        qi = lax.broadcasted_iota(jnp.int32, (SQ, SKV_EFF), 0)
        ki = lax.broadcasted_iota(jnp.int32, (SQ, SKV_EFF), 1)
        mask = ki <= qi + WINDOW

        w_all = [[None] * H_LOC for _ in range(B)]
        for b in range(B):
            for h in range(H_LOC):
                kc = kbuf[:, b, :, h * DH:(h + 1) * DH].reshape(
                    SKV_EFF, DH).astype(jnp.float32)
                scores = lax.dot_general(
                    q_all[b][h], kc, (((1,), (1,)), ((), ())),
                    preferred_element_type=jnp.float32) * 0.125
                scores = jnp.where(mask, scores, NEG)
                m = jnp.max(scores, axis=1, keepdims=True)
                w = jnp.exp(scores - m)
                w_all[b][h] = w / jnp.sum(w, axis=1, keepdims=True)

        for s in range(KV_SRC):
            @pl.when(my != s)
            def _(s=s):
                pltpu.make_async_remote_copy(
                    src_ref=k_ref.at[:, :, pl.ds(0, D_LOC)],
                    dst_ref=vbuf.at[s],
                    send_sem=kv_send_sems.at[0, 1],
                    recv_sem=kv_recv_sems.at[s, 1],
                    device_id=(s,),
                    device_id_type=pl.DeviceIdType.MESH,
                ).wait_recv()

        for b in range(B):
            acc = jnp.zeros((SQ, D_MODEL), jnp.float32)
            for h in range(H_LOC):
                vc = vbuf[:, b, :, h * DH:(h + 1) * DH].reshape(
                    SKV_EFF, DH).astype(jnp.float32)
                ctx = jnp.dot(w_all[b][h], vc,
                              preferred_element_type=jnp.float32)
                acc = acc + jnp.dot(ctx, wo_ref[h * DH:(h + 1) * DH, :],
                                    preferred_element_type=jnp.float32)
            mypart[b] = acc.astype(jnp.bfloat16)
            for d in range(N_DEV):
                @pl.when(my == d)
                def _(d=d, b=b, acc=acc):
                    rsbuf[d, b] = acc[:, d * D_CH:(d + 1) * D_CH].astype(
                        jnp.bfloat16)
                    for j in range(N_DEV):
                        if j == d:
                            continue
                        pltpu.make_async_remote_copy(
                            src_ref=mypart.at[b, :, pl.ds(j * D_CH, D_CH)],
                            dst_ref=rsbuf.at[d, b],
                            send_sem=rs_send_sems.at[j, b],
                            recv_sem=rs_recv_sems.at[d, b],
                            device_id=(j,),
                            device_id_type=pl.DeviceIdType.MESH,
                        ).start()

        for s in range(KV_SRC):
            @pl.when(my == s)
            def _(s=s):
                for d in range(N_DEV):
                    if d == s:
                        continue
                    for t, (src, dst) in enumerate(((k_ref, kbuf), (v_ref, vbuf))):
                        pltpu.make_async_remote_copy(
                            src_ref=src.at[:, :, pl.ds(d * D_LOC, D_LOC)],
                            dst_ref=dst.at[s],
                            send_sem=kv_send_sems.at[d, t],
                            recv_sem=kv_recv_sems.at[s, t],
                            device_id=(d,),
                            device_id_type=pl.DeviceIdType.MESH,
                        ).wait_send()

        for b in range(B):
            for s in range(N_DEV):
                @pl.when(my != s)
                def _(s=s, b=b):
                    pltpu.make_async_remote_copy(
                        src_ref=mypart.at[b, :, pl.ds(0, D_CH)],
                        dst_ref=rsbuf.at[s, b],
                        send_sem=rs_send_sems.at[s, b],
                        recv_sem=rs_recv_sems.at[s, b],
                        device_id=(s,),
                        device_id_type=pl.DeviceIdType.MESH,
                    ).wait_recv()
            red = (rsbuf[0, b].astype(jnp.float32)
                   + rsbuf[1, b].astype(jnp.float32)
                   + rsbuf[2, b].astype(jnp.float32)
                   + rsbuf[3, b].astype(jnp.float32))
            for j in range(N_DEV):
                @pl.when(my == j)
                def _(j=j, b=b, red=red):
                    agbuf[j, b] = red.astype(jnp.bfloat16)
                    for peer in range(N_DEV):
                        if peer == j:
                            continue
                        pltpu.make_async_remote_copy(
                            src_ref=agbuf.at[j, b],
                            dst_ref=agbuf.at[j, b],
                            send_sem=ag_send_sems.at[peer, b],
                            recv_sem=ag_recv_sems.at[j, b],
                            device_id=(peer,),
                            device_id_type=pl.DeviceIdType.MESH,
                        ).start()

        for b in range(B):
            for j in range(N_DEV):
                @pl.when(my != j)
                def _(j=j, b=b):
                    pltpu.make_async_remote_copy(
                        src_ref=agbuf.at[j, b], dst_ref=agbuf.at[j, b],
                        send_sem=ag_send_sems.at[j, b],
                        recv_sem=ag_recv_sems.at[j, b],
                        device_id=(j,),
                        device_id_type=pl.DeviceIdType.MESH,
                    ).wait_recv()
            for j in range(N_DEV):
                out_ref[b, :, j * D_CH:(j + 1) * D_CH] = agbuf[j, b].astype(
                    jnp.float32)

        for d in range(N_DEV):
            @pl.when(my == d)
            def _(d=d):
                for b in range(B):
                    for j in range(N_DEV):
                        if j == d:
                            continue
                        pltpu.make_async_remote_copy(
                            src_ref=mypart.at[b, :, pl.ds(j * D_CH, D_CH)],
                            dst_ref=rsbuf.at[d, b],
                            send_sem=rs_send_sems.at[j, b],
                            recv_sem=rs_recv_sems.at[d, b],
                            device_id=(j,),
                            device_id_type=pl.DeviceIdType.MESH,
                        ).wait_send()
                        pltpu.make_async_remote_copy(
                            src_ref=agbuf.at[d, b], dst_ref=agbuf.at[d, b],
                            send_sem=ag_send_sems.at[j, b],
                            recv_sem=ag_recv_sems.at[d, b],
                            device_id=(j,),
                            device_id_type=pl.DeviceIdType.MESH,
                        ).wait_send()

    return pl.pallas_call(
        body,
        out_shape=jax.ShapeDtypeStruct((B, SQ, D_MODEL), jnp.float32),
        in_specs=[
            pl.BlockSpec(memory_space=pltpu.VMEM),
            pl.BlockSpec(memory_space=pltpu.VMEM),
            pl.BlockSpec(memory_space=pltpu.MemorySpace.HBM),
            pl.BlockSpec(memory_space=pltpu.MemorySpace.HBM),
            pl.BlockSpec(memory_space=pltpu.VMEM),
        ],
        out_specs=pl.BlockSpec(memory_space=pltpu.VMEM),
        scratch_shapes=[
            pltpu.VMEM((KV_SRC, B, SKV_LOC, D_LOC), jnp.bfloat16),
            pltpu.VMEM((KV_SRC, B, SKV_LOC, D_LOC), jnp.bfloat16),
            pltpu.VMEM((B, SQ, D_MODEL), jnp.bfloat16),
            pltpu.VMEM((N_DEV, B, SQ, D_CH), jnp.bfloat16),
            pltpu.VMEM((N_DEV, B, SQ, D_CH), jnp.bfloat16),
            pltpu.SemaphoreType.DMA((N_DEV, 2)),
            pltpu.SemaphoreType.DMA((KV_SRC, 2)),
            pltpu.SemaphoreType.DMA((2,)),
            pltpu.SemaphoreType.DMA((N_DEV, B)),
            pltpu.SemaphoreType.DMA((N_DEV, B)),
            pltpu.SemaphoreType.DMA((N_DEV, B)),
            pltpu.SemaphoreType.DMA((N_DEV, B)),
        ],
        compiler_params=pltpu.CompilerParams(collective_id=0),
    )(x, Wq, K2, V2, Wo)
